# baseline (speedup 1.0000x reference)
import functools

import jax
import jax.numpy as jnp
from jax import lax
from jax.experimental import pallas as pl
from jax.experimental.pallas import tpu as pltpu
from jax.experimental.pallas import tpu_sc as plsc

N_NODES = 10000
N_EDGES = 160000
D_NODE = 128

NC = 2
NS = 16
NW = NC * NS

N_HALF = 2
EH = N_EDGES // N_HALF


G_CH = 128
G_CHN = EH // G_CH
G_MIN = G_CHN // NW
G_EXTRA = G_CHN - G_MIN * NW

_gather_mesh = plsc.VectorSubcoreMesh(
    core_axis_name="c", subcore_axis_name="s", num_cores=NC, num_subcores=NS)


def _make_gather(half):
    @functools.partial(
        pl.kernel,
        out_type=jax.ShapeDtypeStruct((EH, D_NODE), jnp.float32),
        mesh=_gather_mesh,
        scratch_types=[
            pltpu.VMEM((2, G_CH), jnp.int32),
            pltpu.VMEM((G_CH, D_NODE), jnp.float32),
            pltpu.VMEM((G_CH, D_NODE), jnp.float32),
            pltpu.SemaphoreType.DMA,
            pltpu.SemaphoreType.DMA,
        ],
        name=f"sc_gather_h{half}",
    )
    def _sc_gather(nodes_hbm, ei4_hbm, out_hbm, idx2, b0, b1, s0, s1):
        c = lax.axis_index("c")
        s = lax.axis_index("s")
        wid = s * NC + c
        bufs = (b0, b1)
        sems = (s0, s1)
        count = G_MIN + jnp.where(wid < G_EXTRA, 1, 0)

        pltpu.sync_copy(ei4_hbm.at[0, half, wid], idx2.at[0])
        pltpu.sync_copy(ei4_hbm.at[0, half, wid + NW], idx2.at[1])
        pltpu.async_copy(nodes_hbm.at[idx2.at[0]], b0, s0)
        pltpu.async_copy(nodes_hbm.at[idx2.at[1]], b1, s1)

        def pair(t, carry):
            for b in range(2):
                j = 2 * t + b
                buf, sem = bufs[b], sems[b]

                @pl.when(j < count)
                def _chunk():
                    pltpu.make_async_copy(
                        nodes_hbm.at[idx2.at[b]], buf, sem).wait()
                    pltpu.sync_copy(
                        buf,
                        out_hbm.at[pl.ds((wid + NW * j) * G_CH, G_CH), :])
                    nxt = j + 2

                    @pl.when(nxt < count)
                    def _pf():
                        pltpu.sync_copy(
                            ei4_hbm.at[0, half, wid + NW * nxt], idx2.at[b])
                        pltpu.async_copy(nodes_hbm.at[idx2.at[b]], buf, sem)
            return carry

        lax.fori_loop(0, (G_MIN + 2) // 2, pair, 0)

    return _sc_gather


_GATHER = {h: _make_gather(h) for h in range(N_HALF)}


BE = 2000


def _tc_body(fw_ref, fe_ref, x_ref, W1_ref, b1_ref, W2_ref, b2_ref,
             W3_ref, b3_ref, m0_ref, m1_ref, m2_ref, m3_ref):
    fw = fw_ref[...]
    h = jax.nn.silu(jnp.dot(fw, W1_ref[...], preferred_element_type=jnp.float32)
                    + b1_ref[...])
    h = jax.nn.silu(jnp.dot(h, W2_ref[...], preferred_element_type=jnp.float32)
                    + b2_ref[...])
    w = jnp.dot(h, W3_ref[...], preferred_element_type=jnp.float32) + b3_ref[...]
    x = x_ref[...]
    u0 = w[:, :D_NODE] * x
    u1 = w[:, D_NODE:] * x
    fe = fe_ref[...]
    m0_ref[...] = u0 * fe[:, 0:1]
    m1_ref[...] = u1 * fe[:, 1:2]
    m2_ref[...] = u1 * fe[:, 2:3]
    m3_ref[...] = u1 * fe[:, 3:4]


def _tc_messages(half, fw, fe, x_src, W1, b1, W2, b2, W3, b3):
    n_blocks = EH // BE
    off = half * n_blocks
    full = lambda shape: pl.BlockSpec(shape, lambda i: (0, 0))
    hblk = lambda cols: pl.BlockSpec((BE, cols), lambda i: (i + off, 0))
    blk = lambda cols: pl.BlockSpec((BE, cols), lambda i: (i, 0))
    out = pl.pallas_call(
        _tc_body,
        grid=(n_blocks,),
        in_specs=[
            hblk(16), hblk(4), blk(D_NODE),
            full((16, 64)), full((1, 64)),
            full((64, 64)), full((1, 64)),
            full((64, 256)), full((1, 256)),
        ],
        out_specs=[blk(D_NODE)] * 4,
        out_shape=[jax.ShapeDtypeStruct((EH, D_NODE), jnp.float32)] * 4,
    )(fw, fe, x_src, W1, b1.reshape(1, 64), W2, b2.reshape(1, 64),
      W3, b3.reshape(1, 256))
    return out


EPT = EH // NS
S_CH = 40
S_NCH = EPT // S_CH
RPT = N_NODES // NS
ZB = 25

_scatter_mesh = plsc.VectorSubcoreMesh(
    core_axis_name="c", subcore_axis_name="s", num_cores=NC, num_subcores=NS)


def _make_scatter(half):
    @functools.partial(
        pl.kernel,
        out_type=jax.ShapeDtypeStruct((N_NODES, 4, D_NODE), jnp.float32),
        mesh=_scatter_mesh,
        scratch_types=[
            pltpu.VMEM((S_NCH, S_CH), jnp.int32),
            pltpu.VMEM((S_CH, D_NODE), jnp.float32),
            pltpu.VMEM((S_CH, D_NODE), jnp.float32),
            pltpu.VMEM((ZB, D_NODE), jnp.float32),
            pltpu.VMEM_SHARED((N_NODES, D_NODE), jnp.float32),
            pltpu.SemaphoreType.DMA,
            pltpu.SemaphoreType.DMA,
        ],
        name=f"sc_scatter_h{half}",
    )
    def _sc_scatter(m0, m1, m2, m3, ei5_hbm, out_hbm,
                    idx_all, mb0, mb1, zbuf, slab, sm0, sm1):
        c = lax.axis_index("c")
        tid = lax.axis_index("s")

        def zrow(i, carry):
            r = i // 8
            col = (i % 8) * 16
            zbuf[r, pl.ds(col, 16)] = jnp.zeros((16,), jnp.float32)
            return carry

        lax.fori_loop(0, ZB * 8, zrow, 0)

        pltpu.sync_copy(ei5_hbm.at[1, half, tid], idx_all)

        bufs = (mb0, mb1)
        sems = (sm0, sm1)
        planes = (m0, m1, m2, m3)
        for c_val in (0, 1):
            @pl.when(c == c_val)
            def _core():
                for kk in (0, 1):
                    k = 2 * c_val + kk
                    msrc = planes[k]
                    ebase = tid * EPT
                    for z in range(RPT // ZB):
                        pltpu.sync_copy(
                            zbuf, slab.at[pl.ds(tid * RPT + z * ZB, ZB), :])
                    plsc.subcore_barrier()

                    pltpu.async_copy(msrc.at[pl.ds(ebase, S_CH), :], mb0, sm0)
                    pltpu.async_copy(
                        msrc.at[pl.ds(ebase + S_CH, S_CH), :], mb1, sm1)

                    def pair(t, carry):
                        for b in range(2):
                            j = 2 * t + b
                            buf, sem = bufs[b], sems[b]

                            @pl.when(j < S_NCH)
                            def _chunk():
                                pltpu.make_async_copy(
                                    msrc.at[pl.ds(0, S_CH), :],
                                    buf, sem).wait()
                                pltpu.sync_copy(
                                    buf, slab.at[idx_all.at[j]], add=True)
                                nxt = j + 2

                                @pl.when(nxt < S_NCH)
                                def _pf():
                                    pltpu.async_copy(
                                        msrc.at[
                                            pl.ds(ebase + nxt * S_CH, S_CH), :],
                                        buf, sem)
                        return carry

                    lax.fori_loop(0, (S_NCH + 1) // 2, pair, 0)
                    plsc.subcore_barrier()
                    pltpu.sync_copy(
                        slab.at[pl.ds(tid * RPT, RPT), :],
                        out_hbm.at[pl.ds(tid * RPT, RPT), k, :])
                    plsc.subcore_barrier()

    return _sc_scatter


_SCATTER = {h: _make_scatter(h) for h in range(N_HALF)}


BN = 2000


def _asm_body(accA_ref, accB_ref, p1_ref, p2_ref, p3_ref, o_ref):
    o_ref[:, :D_NODE] = accA_ref[:, 0, :] + accB_ref[:, 0, :]
    dots = []
    for j, p_ref in ((1, p1_ref), (2, p2_ref), (3, p3_ref)):
        a = accA_ref[:, j, :] + accB_ref[:, j, :]
        dots.append(jnp.dot(a, p_ref[...],
                            preferred_element_type=jnp.float32,
                            precision=jax.lax.Precision.HIGHEST))
    o_ref[:, D_NODE:] = dots[0] + dots[1] + dots[2]


def _tc_assemble(accA, accB, p1, p2, p3):
    return pl.pallas_call(
        _asm_body,
        grid=(N_NODES // BN,),
        in_specs=[
            pl.BlockSpec((BN, 4, D_NODE), lambda i: (i, 0, 0)),
            pl.BlockSpec((BN, 4, D_NODE), lambda i: (i, 0, 0)),
            pl.BlockSpec((D_NODE, 3 * D_NODE), lambda i: (0, 0)),
            pl.BlockSpec((D_NODE, 3 * D_NODE), lambda i: (0, 0)),
            pl.BlockSpec((D_NODE, 3 * D_NODE), lambda i: (0, 0)),
        ],
        out_specs=pl.BlockSpec((BN, 4 * D_NODE), lambda i: (i, 0)),
        out_shape=jax.ShapeDtypeStruct((N_NODES, 4 * D_NODE), jnp.float32),
    )(accA, accB, p1, p2, p3)


def kernel(features_node, features_edge, features_weights, edge_index,
           W1, b1, W2, b2, W3, b3):
    ei4g = edge_index.reshape(2, N_HALF, G_CHN, G_CH)
    ei5 = edge_index.reshape(2, N_HALF, NS, S_NCH, S_CH)
    xs = [_GATHER[h](features_node, ei4g) for h in range(N_HALF)]
    ms = [_tc_messages(h, features_weights, features_edge, xs[h],
                       W1, b1, W2, b2, W3, b3) for h in range(N_HALF)]
    accs = [_SCATTER[h](*ms[h], ei5) for h in range(N_HALF)]
    ch = jnp.arange(D_NODE)
    perms = []
    for j in range(3):
        p = jnp.zeros((D_NODE, 3 * D_NODE), jnp.float32)
        perms.append(p.at[ch, 3 * ch + j].set(1.0))
    return _tc_assemble(accs[0], accs[1], *perms)


# speedup vs baseline: 1.1801x; 1.1801x over previous
import functools

import numpy as np

import jax
import jax.numpy as jnp
from jax import lax
from jax.experimental import pallas as pl
from jax.experimental.pallas import tpu as pltpu
from jax.experimental.pallas import tpu_sc as plsc

N_NODES = 10000
N_EDGES = 160000
D_NODE = 128

NC = 2
NS = 16
NW = NC * NS

N_HALF = 2
EH = N_EDGES // N_HALF


G_CH = 128
G_CHN = EH // G_CH
G_MIN = G_CHN // NW
G_EXTRA = G_CHN - G_MIN * NW

_gather_mesh = plsc.VectorSubcoreMesh(
    core_axis_name="c", subcore_axis_name="s", num_cores=NC, num_subcores=NS)


def _make_gather(half):
    @functools.partial(
        pl.kernel,
        out_type=jax.ShapeDtypeStruct((EH, D_NODE), jnp.float32),
        mesh=_gather_mesh,
        scratch_types=[
            pltpu.VMEM((2, G_CH), jnp.int32),
            pltpu.VMEM((G_CH, D_NODE), jnp.float32),
            pltpu.VMEM((G_CH, D_NODE), jnp.float32),
            pltpu.SemaphoreType.DMA,
            pltpu.SemaphoreType.DMA,
        ],
        name=f"sc_gather_h{half}",
    )
    def _sc_gather(nodes_hbm, ei4_hbm, out_hbm, idx2, b0, b1, s0, s1):
        c = lax.axis_index("c")
        s = lax.axis_index("s")
        wid = s * NC + c
        bufs = (b0, b1)
        sems = (s0, s1)
        count = G_MIN + jnp.where(wid < G_EXTRA, 1, 0)

        pltpu.sync_copy(ei4_hbm.at[0, half, wid], idx2.at[0])
        pltpu.sync_copy(ei4_hbm.at[0, half, wid + NW], idx2.at[1])
        pltpu.async_copy(nodes_hbm.at[idx2.at[0]], b0, s0)
        pltpu.async_copy(nodes_hbm.at[idx2.at[1]], b1, s1)

        def pair(t, carry):
            for b in range(2):
                j = 2 * t + b
                buf, sem = bufs[b], sems[b]

                @pl.when(j < count)
                def _chunk():
                    pltpu.make_async_copy(
                        nodes_hbm.at[idx2.at[b]], buf, sem).wait()
                    pltpu.sync_copy(
                        buf,
                        out_hbm.at[pl.ds((wid + NW * j) * G_CH, G_CH), :])
                    nxt = j + 2

                    @pl.when(nxt < count)
                    def _pf():
                        pltpu.sync_copy(
                            ei4_hbm.at[0, half, wid + NW * nxt], idx2.at[b])
                        pltpu.async_copy(nodes_hbm.at[idx2.at[b]], buf, sem)
            return carry

        lax.fori_loop(0, (G_MIN + 2) // 2, pair, 0)

    return _sc_gather


_GATHER = {h: _make_gather(h) for h in range(N_HALF)}


BE = 2000


def _tc_body(fw_ref, fe_ref, x_ref, W1_ref, b1_ref, W2_ref, b2_ref,
             W3_ref, b3_ref, m0_ref, m1_ref, m2_ref, m3_ref):
    fw = fw_ref[...]
    h = jax.nn.silu(jnp.dot(fw, W1_ref[...], preferred_element_type=jnp.float32)
                    + b1_ref[...])
    h = jax.nn.silu(jnp.dot(h, W2_ref[...], preferred_element_type=jnp.float32)
                    + b2_ref[...])
    w = jnp.dot(h, W3_ref[...], preferred_element_type=jnp.float32) + b3_ref[...]
    x = x_ref[...]
    u0 = w[:, :D_NODE] * x
    u1 = w[:, D_NODE:] * x
    fe = fe_ref[...]
    m0_ref[...] = u0 * fe[:, 0:1]
    m1_ref[...] = u1 * fe[:, 1:2]
    m2_ref[...] = u1 * fe[:, 2:3]
    m3_ref[...] = u1 * fe[:, 3:4]


def _tc_messages(half, fw, fe, x_src, W1, b1, W2, b2, W3, b3):
    n_blocks = EH // BE
    off = half * n_blocks
    full = lambda shape: pl.BlockSpec(shape, lambda i: (0, 0))
    hblk = lambda cols: pl.BlockSpec((BE, cols), lambda i: (i + off, 0))
    blk = lambda cols: pl.BlockSpec((BE, cols), lambda i: (i, 0))
    out = pl.pallas_call(
        _tc_body,
        grid=(n_blocks,),
        in_specs=[
            hblk(16), hblk(4), blk(D_NODE),
            full((16, 64)), full((1, 64)),
            full((64, 64)), full((1, 64)),
            full((64, 256)), full((1, 256)),
        ],
        out_specs=[blk(D_NODE)] * 4,
        out_shape=[jax.ShapeDtypeStruct((EH, D_NODE), jnp.float32)] * 4,
    )(fw, fe, x_src, W1, b1.reshape(1, 64), W2, b2.reshape(1, 64),
      W3, b3.reshape(1, 256))
    return out


S_CH = 128
S_CHN = EH // S_CH
S_MIN = S_CHN // NS
S_EXTRA = S_CHN - S_MIN * NS
RPT = N_NODES // NS
ZB = 25

_scatter_mesh = plsc.VectorSubcoreMesh(
    core_axis_name="c", subcore_axis_name="s", num_cores=NC, num_subcores=NS)


def _make_scatter(half):
    @functools.partial(
        pl.kernel,
        out_type=jax.ShapeDtypeStruct((N_NODES, 4, D_NODE), jnp.float32),
        mesh=_scatter_mesh,
        scratch_types=[
            pltpu.VMEM((2, S_CH), jnp.int32),
            pltpu.VMEM((S_CH, D_NODE), jnp.float32),
            pltpu.VMEM((S_CH, D_NODE), jnp.float32),
            pltpu.VMEM((ZB, D_NODE), jnp.float32),
            pltpu.VMEM_SHARED((N_NODES, D_NODE), jnp.float32),
            pltpu.SemaphoreType.DMA,
            pltpu.SemaphoreType.DMA,
        ],
        name=f"sc_scatter_h{half}",
    )
    def _sc_scatter(m0, m1, m2, m3, ei4_hbm, out_hbm,
                    idx2, mb0, mb1, zbuf, slab, sm0, sm1):
        c = lax.axis_index("c")
        tid = lax.axis_index("s")
        count = S_MIN + jnp.where(tid < S_EXTRA, 1, 0)

        def zrow(i, carry):
            r = i // 8
            col = (i % 8) * 16
            zbuf[r, pl.ds(col, 16)] = jnp.zeros((16,), jnp.float32)
            return carry

        lax.fori_loop(0, ZB * 8, zrow, 0)

        bufs = (mb0, mb1)
        sems = (sm0, sm1)
        planes = (m0, m1, m2, m3)
        for c_val in (0, 1):
            @pl.when(c == c_val)
            def _core():
                for kk in (0, 1):
                    k = 2 * c_val + kk
                    msrc = planes[k]
                    for z in range(RPT // ZB):
                        pltpu.sync_copy(
                            zbuf, slab.at[pl.ds(tid * RPT + z * ZB, ZB), :])
                    plsc.subcore_barrier()

                    pltpu.sync_copy(ei4_hbm.at[1, half, tid], idx2.at[0])
                    pltpu.sync_copy(ei4_hbm.at[1, half, tid + NS], idx2.at[1])
                    pltpu.async_copy(
                        msrc.at[pl.ds(tid * S_CH, S_CH), :], mb0, sm0)
                    pltpu.async_copy(
                        msrc.at[pl.ds((tid + NS) * S_CH, S_CH), :], mb1, sm1)

                    def pair(t, carry):
                        for b in range(2):
                            j = 2 * t + b
                            buf, sem = bufs[b], sems[b]

                            @pl.when(j < count)
                            def _chunk():
                                pltpu.make_async_copy(
                                    msrc.at[pl.ds(0, S_CH), :],
                                    buf, sem).wait()
                                pltpu.sync_copy(
                                    buf, slab.at[idx2.at[b]], add=True)
                                nxt = j + 2

                                @pl.when(nxt < count)
                                def _pf():
                                    pltpu.sync_copy(
                                        ei4_hbm.at[1, half, tid + NS * nxt],
                                        idx2.at[b])
                                    pltpu.async_copy(
                                        msrc.at[
                                            pl.ds((tid + NS * nxt) * S_CH,
                                                  S_CH), :],
                                        buf, sem)
                        return carry

                    lax.fori_loop(0, (S_MIN + 2) // 2, pair, 0)
                    plsc.subcore_barrier()
                    pltpu.sync_copy(
                        slab.at[pl.ds(tid * RPT, RPT), :],
                        out_hbm.at[pl.ds(tid * RPT, RPT), k, :])
                    plsc.subcore_barrier()

    return _sc_scatter


_SCATTER = {h: _make_scatter(h) for h in range(N_HALF)}


BN = 2000


def _asm_body(accA_ref, accB_ref, p1_ref, p2_ref, p3_ref, o_ref):
    o_ref[:, :D_NODE] = accA_ref[:, 0, :] + accB_ref[:, 0, :]
    dots = []
    for j, p_ref in ((1, p1_ref), (2, p2_ref), (3, p3_ref)):
        a = accA_ref[:, j, :] + accB_ref[:, j, :]
        dots.append(jnp.dot(a, p_ref[...],
                            preferred_element_type=jnp.float32))
    o_ref[:, D_NODE:] = dots[0] + dots[1] + dots[2]


def _tc_assemble(accA, accB, p1, p2, p3):
    return pl.pallas_call(
        _asm_body,
        grid=(N_NODES // BN,),
        in_specs=[
            pl.BlockSpec((BN, 4, D_NODE), lambda i: (i, 0, 0)),
            pl.BlockSpec((BN, 4, D_NODE), lambda i: (i, 0, 0)),
            pl.BlockSpec((D_NODE, 3 * D_NODE), lambda i: (0, 0)),
            pl.BlockSpec((D_NODE, 3 * D_NODE), lambda i: (0, 0)),
            pl.BlockSpec((D_NODE, 3 * D_NODE), lambda i: (0, 0)),
        ],
        out_specs=pl.BlockSpec((BN, 4 * D_NODE), lambda i: (i, 0)),
        out_shape=jax.ShapeDtypeStruct((N_NODES, 4 * D_NODE), jnp.float32),
    )(accA, accB, p1, p2, p3)


def _make_perms():
    perms = []
    for j in range(3):
        p = np.zeros((D_NODE, 3 * D_NODE), np.float32)
        p[np.arange(D_NODE), 3 * np.arange(D_NODE) + j] = 1.0
        perms.append(jnp.asarray(p))
    return perms


_PERMS = _make_perms()


def kernel(features_node, features_edge, features_weights, edge_index,
           W1, b1, W2, b2, W3, b3):
    ei4g = edge_index.reshape(2, N_HALF, G_CHN, G_CH)
    xs = [_GATHER[h](features_node, ei4g) for h in range(N_HALF)]
    ms = [_tc_messages(h, features_weights, features_edge, xs[h],
                       W1, b1, W2, b2, W3, b3) for h in range(N_HALF)]
    accs = [_SCATTER[h](*ms[h], ei4g) for h in range(N_HALF)]
    return _tc_assemble(accs[0], accs[1], *_PERMS)


# speedup vs baseline: 1.1813x; 1.0010x over previous
import functools

import numpy as np

import jax
import jax.numpy as jnp
from jax import lax
from jax.experimental import pallas as pl
from jax.experimental.pallas import tpu as pltpu
from jax.experimental.pallas import tpu_sc as plsc

N_NODES = 10000
N_EDGES = 160000
D_NODE = 128

NC = 2
NS = 16
NW = NC * NS

N_HALF = 2
EH = N_EDGES // N_HALF


G_CH = 128
G_CHN = EH // G_CH
G_MIN = G_CHN // NW
G_EXTRA = G_CHN - G_MIN * NW

_gather_mesh = plsc.VectorSubcoreMesh(
    core_axis_name="c", subcore_axis_name="s", num_cores=NC, num_subcores=NS)


def _make_gather(half):
    @functools.partial(
        pl.kernel,
        out_type=jax.ShapeDtypeStruct((EH, D_NODE), jnp.float32),
        mesh=_gather_mesh,
        scratch_types=[
            pltpu.VMEM((2, G_CH), jnp.int32),
            pltpu.VMEM((G_CH, D_NODE), jnp.float32),
            pltpu.VMEM((G_CH, D_NODE), jnp.float32),
            pltpu.SemaphoreType.DMA,
            pltpu.SemaphoreType.DMA,
        ],
        name=f"sc_gather_h{half}",
    )
    def _sc_gather(nodes_hbm, ei4_hbm, out_hbm, idx2, b0, b1, s0, s1):
        c = lax.axis_index("c")
        s = lax.axis_index("s")
        wid = s * NC + c
        bufs = (b0, b1)
        sems = (s0, s1)
        count = G_MIN + jnp.where(wid < G_EXTRA, 1, 0)

        pltpu.sync_copy(ei4_hbm.at[0, half, wid], idx2.at[0])
        pltpu.sync_copy(ei4_hbm.at[0, half, wid + NW], idx2.at[1])
        pltpu.async_copy(nodes_hbm.at[idx2.at[0]], b0, s0)
        pltpu.async_copy(nodes_hbm.at[idx2.at[1]], b1, s1)

        def pair(t, carry):
            for b in range(2):
                j = 2 * t + b
                buf, sem = bufs[b], sems[b]

                @pl.when(j < count)
                def _chunk():
                    pltpu.make_async_copy(
                        nodes_hbm.at[idx2.at[b]], buf, sem).wait()
                    pltpu.sync_copy(
                        buf,
                        out_hbm.at[pl.ds((wid + NW * j) * G_CH, G_CH), :])
                    nxt = j + 2

                    @pl.when(nxt < count)
                    def _pf():
                        pltpu.sync_copy(
                            ei4_hbm.at[0, half, wid + NW * nxt], idx2.at[b])
                        pltpu.async_copy(nodes_hbm.at[idx2.at[b]], buf, sem)
            return carry

        lax.fori_loop(0, (G_MIN + 2) // 2, pair, 0)

    return _sc_gather


_GATHER = {h: _make_gather(h) for h in range(N_HALF)}


BE = 2000


def _tc_body(fw_ref, fe_ref, x_hbm, W1_ref, b1_ref, W2_ref, b2_ref,
             W3_ref, b3_ref, m0_ref, m1_ref, m2_ref, m3_ref,
             xb0, xb1, sx0, sx1):
    i = pl.program_id(0)
    n = pl.num_programs(0)
    xbufs = (xb0, xb1)
    xsems = (sx0, sx1)

    @pl.when(i == 0)
    def _prime():
        pltpu.make_async_copy(
            x_hbm.at[pl.ds(0, BE), :], xb0, sx0).start()

    @pl.when(i + 1 < n)
    def _prefetch():
        for par in (0, 1):
            @pl.when(lax.rem(i + 1, 2) == par)
            def _():
                pltpu.make_async_copy(
                    x_hbm.at[pl.ds((i + 1) * BE, BE), :],
                    xbufs[par], xsems[par]).start()

    fw = fw_ref[...]
    h = jax.nn.silu(jnp.dot(fw, W1_ref[...], preferred_element_type=jnp.float32)
                    + b1_ref[...])
    h = jax.nn.silu(jnp.dot(h, W2_ref[...], preferred_element_type=jnp.float32)
                    + b2_ref[...])
    w = jnp.dot(h, W3_ref[...], preferred_element_type=jnp.float32) + b3_ref[...]
    for par in (0, 1):
        @pl.when(lax.rem(i, 2) == par)
        def _wait():
            pltpu.make_async_copy(
                x_hbm.at[pl.ds(0, BE), :], xbufs[par], xsems[par]).wait()

    x = jnp.where(lax.rem(i, 2) == 0, xb0[...], xb1[...])
    u0 = w[:, :D_NODE] * x
    u1 = w[:, D_NODE:] * x
    fe = fe_ref[...]
    m0_ref[...] = u0 * fe[:, 0:1]
    m1_ref[...] = u1 * fe[:, 1:2]
    m2_ref[...] = u1 * fe[:, 2:3]
    m3_ref[...] = u1 * fe[:, 3:4]


def _tc_messages(half, fw, fe, x_src, W1, b1, W2, b2, W3, b3):
    n_blocks = EH // BE
    off = half * n_blocks
    full = lambda shape: pl.BlockSpec(shape, lambda i: (0, 0))
    hblk = lambda cols: pl.BlockSpec((BE, cols), lambda i: (i + off, 0))
    blk = lambda cols: pl.BlockSpec((BE, cols), lambda i: (i, 0))
    out = pl.pallas_call(
        _tc_body,
        grid=(n_blocks,),
        in_specs=[
            hblk(16), hblk(4),
            pl.BlockSpec(memory_space=pl.ANY),
            full((16, 64)), full((1, 64)),
            full((64, 64)), full((1, 64)),
            full((64, 256)), full((1, 256)),
        ],
        out_specs=[blk(D_NODE)] * 4,
        out_shape=[jax.ShapeDtypeStruct((EH, D_NODE), jnp.float32)] * 4,
        scratch_shapes=[
            pltpu.VMEM((BE, D_NODE), jnp.float32),
            pltpu.VMEM((BE, D_NODE), jnp.float32),
            pltpu.SemaphoreType.DMA,
            pltpu.SemaphoreType.DMA,
        ],
    )(fw, fe, x_src, W1, b1.reshape(1, 64), W2, b2.reshape(1, 64),
      W3, b3.reshape(1, 256))
    return out


S_CH = 128
S_CHN = EH // S_CH
S_MIN = S_CHN // NS
S_EXTRA = S_CHN - S_MIN * NS
RPT = N_NODES // NS
ZB = 25

_scatter_mesh = plsc.VectorSubcoreMesh(
    core_axis_name="c", subcore_axis_name="s", num_cores=NC, num_subcores=NS)


def _make_scatter(half):
    @functools.partial(
        pl.kernel,
        out_type=jax.ShapeDtypeStruct((N_NODES, 4, D_NODE), jnp.float32),
        mesh=_scatter_mesh,
        scratch_types=[
            pltpu.VMEM((2, S_CH), jnp.int32),
            pltpu.VMEM((S_CH, D_NODE), jnp.float32),
            pltpu.VMEM((S_CH, D_NODE), jnp.float32),
            pltpu.VMEM((ZB, D_NODE), jnp.float32),
            pltpu.VMEM_SHARED((N_NODES, D_NODE), jnp.float32),
            pltpu.SemaphoreType.DMA,
            pltpu.SemaphoreType.DMA,
        ],
        name=f"sc_scatter_h{half}",
    )
    def _sc_scatter(m0, m1, m2, m3, ei4_hbm, out_hbm,
                    idx2, mb0, mb1, zbuf, slab, sm0, sm1):
        c = lax.axis_index("c")
        tid = lax.axis_index("s")
        count = S_MIN + jnp.where(tid < S_EXTRA, 1, 0)

        def zrow(i, carry):
            r = i // 8
            col = (i % 8) * 16
            zbuf[r, pl.ds(col, 16)] = jnp.zeros((16,), jnp.float32)
            return carry

        lax.fori_loop(0, ZB * 8, zrow, 0)

        bufs = (mb0, mb1)
        sems = (sm0, sm1)
        planes = (m0, m1, m2, m3)
        for c_val in (0, 1):
            @pl.when(c == c_val)
            def _core():
                for kk in (0, 1):
                    k = 2 * c_val + kk
                    msrc = planes[k]
                    for z in range(RPT // ZB):
                        pltpu.sync_copy(
                            zbuf, slab.at[pl.ds(tid * RPT + z * ZB, ZB), :])
                    plsc.subcore_barrier()

                    pltpu.sync_copy(ei4_hbm.at[1, half, tid], idx2.at[0])
                    pltpu.sync_copy(ei4_hbm.at[1, half, tid + NS], idx2.at[1])
                    pltpu.async_copy(
                        msrc.at[pl.ds(tid * S_CH, S_CH), :], mb0, sm0)
                    pltpu.async_copy(
                        msrc.at[pl.ds((tid + NS) * S_CH, S_CH), :], mb1, sm1)

                    def pair(t, carry):
                        for b in range(2):
                            j = 2 * t + b
                            buf, sem = bufs[b], sems[b]

                            @pl.when(j < count)
                            def _chunk():
                                pltpu.make_async_copy(
                                    msrc.at[pl.ds(0, S_CH), :],
                                    buf, sem).wait()
                                pltpu.sync_copy(
                                    buf, slab.at[idx2.at[b]], add=True)
                                nxt = j + 2

                                @pl.when(nxt < count)
                                def _pf():
                                    pltpu.sync_copy(
                                        ei4_hbm.at[1, half, tid + NS * nxt],
                                        idx2.at[b])
                                    pltpu.async_copy(
                                        msrc.at[
                                            pl.ds((tid + NS * nxt) * S_CH,
                                                  S_CH), :],
                                        buf, sem)
                        return carry

                    lax.fori_loop(0, (S_MIN + 2) // 2, pair, 0)
                    plsc.subcore_barrier()
                    pltpu.sync_copy(
                        slab.at[pl.ds(tid * RPT, RPT), :],
                        out_hbm.at[pl.ds(tid * RPT, RPT), k, :])
                    plsc.subcore_barrier()

    return _sc_scatter


_SCATTER = {h: _make_scatter(h) for h in range(N_HALF)}


BN = 2000


def _asm_body(accA_ref, accB_ref, p1_ref, p2_ref, p3_ref, o_ref):
    o_ref[:, :D_NODE] = accA_ref[:, 0, :] + accB_ref[:, 0, :]
    dots = []
    for j, p_ref in ((1, p1_ref), (2, p2_ref), (3, p3_ref)):
        a = accA_ref[:, j, :] + accB_ref[:, j, :]
        dots.append(jnp.dot(a, p_ref[...],
                            preferred_element_type=jnp.float32))
    o_ref[:, D_NODE:] = dots[0] + dots[1] + dots[2]


def _tc_assemble(accA, accB, p1, p2, p3):
    return pl.pallas_call(
        _asm_body,
        grid=(N_NODES // BN,),
        in_specs=[
            pl.BlockSpec((BN, 4, D_NODE), lambda i: (i, 0, 0)),
            pl.BlockSpec((BN, 4, D_NODE), lambda i: (i, 0, 0)),
            pl.BlockSpec((D_NODE, 3 * D_NODE), lambda i: (0, 0)),
            pl.BlockSpec((D_NODE, 3 * D_NODE), lambda i: (0, 0)),
            pl.BlockSpec((D_NODE, 3 * D_NODE), lambda i: (0, 0)),
        ],
        out_specs=pl.BlockSpec((BN, 4 * D_NODE), lambda i: (i, 0)),
        out_shape=jax.ShapeDtypeStruct((N_NODES, 4 * D_NODE), jnp.float32),
    )(accA, accB, p1, p2, p3)


def _make_perms():
    perms = []
    for j in range(3):
        p = np.zeros((D_NODE, 3 * D_NODE), np.float32)
        p[np.arange(D_NODE), 3 * np.arange(D_NODE) + j] = 1.0
        perms.append(jnp.asarray(p))
    return perms


_PERMS = _make_perms()


def kernel(features_node, features_edge, features_weights, edge_index,
           W1, b1, W2, b2, W3, b3):
    ei4g = edge_index.reshape(2, N_HALF, G_CHN, G_CH)
    xs = [_GATHER[h](features_node, ei4g) for h in range(N_HALF)]
    ms = [_tc_messages(h, features_weights, features_edge, xs[h],
                       W1, b1, W2, b2, W3, b3) for h in range(N_HALF)]
    accs = [_SCATTER[h](*ms[h], ei4g) for h in range(N_HALF)]
    return _tc_assemble(accs[0], accs[1], *_PERMS)
